# baseline (speedup 1.0000x reference)
import jax
import jax.numpy as jnp
from jax.experimental import pallas as pl
from jax.experimental.pallas import tpu as pltpu

NUM_HOP = 3


def _attn_encoder(x, mask, att_W, att_v):
    s = jnp.tanh(x @ att_W) @ att_v
    s = jnp.where(mask > 0, s, -1e9)
    a = jax.nn.softmax(s, axis=-1)
    return jnp.sum(a[..., None] * x, axis=1)


def _final_stage_kernel(score_ref, ans_ref, loss_ref, pred_ref, dist_ref):
    x = score_ref[...]
    y = ans_ref[...].astype(jnp.float32)
    loss = jnp.mean(jnp.maximum(x, 0.0) - x * y + jnp.log1p(jnp.exp(-jnp.abs(x))))
    loss_ref[...] = jnp.full((1, 1), loss, dtype=jnp.float32)
    pred_ref[...] = jnp.argmax(x, axis=1).astype(jnp.int32)[:, None]
    dist_ref[...] = jax.nn.sigmoid(x)


def kernel(questions, key_kb, rel_word_ids, key_doc, val_kb, val_doc, candidate_entities, answers, entity_emb, word_emb, entity_linear_W, entity_linear_b, att_W, att_v, rel_W, rel_b, query_W, query_b, key_kb_W, key_kb_b, key_doc_W, key_doc_b, value_W, value_b, out_W, out_b):
    lrelu = jax.nn.leaky_relu
    bsz = key_kb.shape[0]; num_mem = key_kb.shape[1]
    emb_q = jnp.take(word_emb, questions, axis=0)
    mask_q = (questions != 1).astype(jnp.float32)
    enc_q = jnp.tanh(emb_q @ query_W + query_b) * mask_q[..., None]
    encoded_q = jnp.sum(enc_q, axis=1, keepdims=True)
    kk_sub = key_kb[:, :, 0]; kk_rel = key_kb[:, :, 1]
    sub_emb = lrelu(jnp.take(entity_emb, kk_sub, axis=0) @ entity_linear_W + entity_linear_b)
    rel_word_emb = jnp.take(word_emb, rel_word_ids, axis=0)
    rel_mask = (rel_word_ids != 1).astype(jnp.float32)
    rel_agg = _attn_encoder(rel_word_emb, rel_mask, att_W, att_v)
    rel_encoded = jnp.tanh(rel_agg @ rel_W + rel_b)
    kk_rel_emb = jnp.take(rel_encoded, kk_rel.reshape(-1), axis=0).reshape(bsz, num_mem, -1)[:, :, None, :]
    emb_key_kb = jnp.concatenate([sub_emb[:, :, None, :], kk_rel_emb], axis=2)
    mask_key_kb = (key_kb != 0).astype(jnp.float32)
    enc_kk = jnp.tanh(emb_key_kb @ key_kb_W + key_kb_b) * mask_key_kb[..., None]
    encoded_key_kb = jnp.sum(enc_kk, axis=2)
    emb_kd = jnp.take(word_emb, key_doc, axis=0)
    mask_kd = (key_doc != 1).astype(jnp.float32)
    enc_kd = jnp.tanh(emb_kd @ key_doc_W + key_doc_b) * mask_kd[..., None]
    encoded_key_doc = jnp.sum(enc_kd, axis=2)
    encoded_key = jnp.concatenate([encoded_key_kb, encoded_key_doc], axis=1)
    emb_vk = jnp.take(entity_emb, val_kb, axis=0)
    mask_vk = (val_kb != 0).astype(jnp.float32)
    enc_vk = jnp.tanh(emb_vk @ value_W + value_b) * mask_vk[..., None]
    emb_vd = jnp.take(entity_emb, val_doc, axis=0)
    mask_vd = (val_doc != 0).astype(jnp.float32)
    enc_vd = jnp.tanh(emb_vd @ value_W + value_b) * mask_vd[..., None]
    encoded_value = jnp.concatenate([enc_vk, enc_vd], axis=1)
    emb_c = jnp.take(entity_emb, candidate_entities, axis=0)
    encoded_candidates = lrelu(emb_c @ entity_linear_W + entity_linear_b)
    for _ in range(NUM_HOP):
        ph = jnp.einsum('bqh,bmh->bqm', encoded_q, encoded_key)
        ph = jnp.where(ph == 0.0, -1e9, ph)
        score = jax.nn.softmax(ph, axis=-1)
        out = jnp.einsum('bqm,bmh->bqh', score, encoded_value)
        encoded_q = (encoded_q + out) @ out_W + out_b
    score_pred = jnp.sum(encoded_q * encoded_candidates, axis=2)
    loss, pred, pred_dist = pl.pallas_call(
        _final_stage_kernel,
        out_shape=(
            jax.ShapeDtypeStruct((1, 1), jnp.float32),
            jax.ShapeDtypeStruct((score_pred.shape[0], 1), jnp.int32),
            jax.ShapeDtypeStruct(score_pred.shape, jnp.float32),
        ),
    )(score_pred, answers)
    return (loss[0, 0], pred[:, 0], pred_dist)


# speedup vs baseline: 1.1208x; 1.1208x over previous
import jax
import jax.numpy as jnp
from jax import lax
from jax.experimental import pallas as pl
from jax.experimental.pallas import tpu as pltpu
from jax.experimental.pallas import tpu_sc as plsc

B = 1024
QL = 20
M = 100
MD = 50
DL = 20
RL = 10
NR = 1000
NENT = 1000000
NWRD = 100000
D = 64
C = 200
NUM_HOP = 3

NC = 2
NS = 16
NWK = NC * NS

_MESH = plsc.VectorSubcoreMesh(core_axis_name="c", subcore_axis_name="s")


def _wid():
    return lax.axis_index("s") * NC + lax.axis_index("c")


_A_G1 = 491520
_A_PW1 = _A_G1 // NWK
_A_R = 256
_A_NCH = _A_PW1 // _A_R
_A_G2 = 12288
_A_PW2 = _A_G2 // NWK


def _sc_a_body(ent_hbm, wrd_hbm, eidx_hbm, widx_hbm, ent_out, wrd_out,
               idx0, idx1, rows0, rows1, sem0, sem1):
    wid = _wid()
    bufs = ((idx0, rows0, sem0), (idx1, rows1, sem1))

    def fire(c, par):
        idxb, rowsb, semb = bufs[par]
        pltpu.sync_copy(eidx_hbm.at[pl.ds(wid * _A_PW1 + c * _A_R, _A_R)], idxb)
        for k in range(2):
            pltpu.async_copy(ent_hbm.at[idxb.at[pl.ds(k * 128, 128)]],
                             rowsb.at[pl.ds(k * 128, 128)], semb)

    def drain(par):
        idxb, rowsb, semb = bufs[par]
        for k in range(2):
            pltpu.make_async_copy(ent_hbm.at[idxb.at[pl.ds(k * 128, 128)]],
                                  rowsb.at[pl.ds(k * 128, 128)], semb).wait()

    fire(0, 0)
    fire(1, 1)

    @pl.loop(0, _A_NCH, step=2)
    def _(c0):
        for par in range(2):
            c = c0 + par
            idxb, rowsb, semb = bufs[par]
            drain(par)
            pltpu.sync_copy(rowsb, ent_out.at[pl.ds(wid * _A_PW1 + c * _A_R, _A_R)])
            nxt = c + 2

            @pl.when(nxt < _A_NCH)
            def _():
                fire(nxt, par)

    for k in range(3):
        pltpu.sync_copy(widx_hbm.at[pl.ds(wid * _A_PW2 + k * 128, 128)],
                        idx0.at[pl.ds(0, 128)])
        pltpu.async_copy(wrd_hbm.at[idx0.at[pl.ds(0, 128)]],
                         rows0.at[pl.ds(0, 128)], sem0)
        pltpu.make_async_copy(wrd_hbm.at[idx0.at[pl.ds(0, 128)]],
                              rows0.at[pl.ds(0, 128)], sem0).wait()
        pltpu.sync_copy(rows0.at[pl.ds(0, 128)],
                        wrd_out.at[pl.ds(wid * _A_PW2 + k * 128, 128)])


def _sc_a(ent_pairs, wrd_pairs, eidx, widx):
    return pl.kernel(
        _sc_a_body,
        out_type=(jax.ShapeDtypeStruct((_A_G1, 2 * D), jnp.float32),
                  jax.ShapeDtypeStruct((_A_G2, 2 * D), jnp.float32)),
        mesh=_MESH,
        scratch_types=[
            pltpu.VMEM((_A_R,), jnp.int32),
            pltpu.VMEM((_A_R,), jnp.int32),
            pltpu.VMEM((_A_R, 2 * D), jnp.float32),
            pltpu.VMEM((_A_R, 2 * D), jnp.float32),
            pltpu.SemaphoreType.DMA,
            pltpu.SemaphoreType.DMA,
        ],
    )(ent_pairs, wrd_pairs, eidx, widx)


_B_S = 20
_B_G = 53248
_B_PW = _B_G // NWK
_B_CH = 16
_B_R = _B_CH * _B_S
_B_NCH = _B_PW // _B_CH


def _sc_b_body(tab_hbm, idx_hbm, out_hbm,
               idx0, idx1, rows0, rows1, outv, sem0, sem1):
    wid = _wid()
    bufs = ((idx0, rows0, sem0), (idx1, rows1, sem1))
    npw = _B_PW * _B_S
    slc = ((0, 128), (128, 128), (256, 64))

    def fire(c, par):
        idxb, rowsb, semb = bufs[par]
        pltpu.sync_copy(idx_hbm.at[pl.ds(wid * npw + c * _B_R, _B_R)], idxb)
        for o, n in slc:
            pltpu.async_copy(tab_hbm.at[idxb.at[pl.ds(o, n)]],
                             rowsb.at[pl.ds(o, n)], semb)

    def drain(par):
        idxb, rowsb, semb = bufs[par]
        for o, n in slc:
            pltpu.make_async_copy(tab_hbm.at[idxb.at[pl.ds(o, n)]],
                                  rowsb.at[pl.ds(o, n)], semb).wait()

    fire(0, 0)
    fire(1, 1)

    @pl.loop(0, _B_NCH, step=2)
    def _(c0):
        for par in range(2):
            c = c0 + par
            idxb, rowsb, semb = bufs[par]
            drain(par)

            @pl.loop(0, _B_CH)
            def _(g):
                r0 = g * _B_S
                for dd in range(8):
                    sl = pl.ds(dd * 16, 16)
                    acc = rowsb[r0, sl]
                    for s in range(1, _B_S):
                        acc = acc + rowsb[r0 + s, sl]
                    outv[g, sl] = acc

            pltpu.sync_copy(outv, out_hbm.at[pl.ds(wid * _B_PW + c * _B_CH, _B_CH)])
            nxt = c + 2

            @pl.when(nxt < _B_NCH)
            def _():
                fire(nxt, par)


def _sc_b(table, idx):
    return pl.kernel(
        _sc_b_body,
        out_type=jax.ShapeDtypeStruct((_B_G, 2 * D), jnp.float32),
        mesh=_MESH,
        scratch_types=[
            pltpu.VMEM((_B_R,), jnp.int32),
            pltpu.VMEM((_B_R,), jnp.int32),
            pltpu.VMEM((_B_R, 2 * D), jnp.float32),
            pltpu.VMEM((_B_R, 2 * D), jnp.float32),
            pltpu.VMEM((_B_CH, 2 * D), jnp.float32),
            pltpu.SemaphoreType.DMA,
            pltpu.SemaphoreType.DMA,
        ],
    )(table, idx)


_C_G = 122880
_C_PW = _C_G // NWK
_C_R = 384
_C_NCH = _C_PW // _C_R


def _sc_c_body(tab_hbm, idx_hbm, out_hbm, idx0, idx1, rows0, rows1, sem0, sem1):
    wid = _wid()
    bufs = ((idx0, rows0, sem0), (idx1, rows1, sem1))

    def fire(c, par):
        idxb, rowsb, semb = bufs[par]
        pltpu.sync_copy(idx_hbm.at[pl.ds(wid * _C_PW + c * _C_R, _C_R)], idxb)
        for k in range(3):
            pltpu.async_copy(tab_hbm.at[idxb.at[pl.ds(k * 128, 128)]],
                             rowsb.at[pl.ds(k * 128, 128)], semb)

    def drain(par):
        idxb, rowsb, semb = bufs[par]
        for k in range(3):
            pltpu.make_async_copy(tab_hbm.at[idxb.at[pl.ds(k * 128, 128)]],
                                  rowsb.at[pl.ds(k * 128, 128)], semb).wait()

    fire(0, 0)
    fire(1, 1)

    @pl.loop(0, _C_NCH, step=2)
    def _(c0):
        for par in range(2):
            c = c0 + par
            idxb, rowsb, semb = bufs[par]
            drain(par)
            pltpu.sync_copy(rowsb, out_hbm.at[pl.ds(wid * _C_PW + c * _C_R, _C_R)])
            nxt = c + 2

            @pl.when(nxt < _C_NCH)
            def _():
                fire(nxt, par)


def _sc_c(table, idx):
    return pl.kernel(
        _sc_c_body,
        out_type=jax.ShapeDtypeStruct((_C_G, 2 * D), jnp.float32),
        mesh=_MESH,
        scratch_types=[
            pltpu.VMEM((_C_R,), jnp.int32),
            pltpu.VMEM((_C_R,), jnp.int32),
            pltpu.VMEM((_C_R, 2 * D), jnp.float32),
            pltpu.VMEM((_C_R, 2 * D), jnp.float32),
            pltpu.SemaphoreType.DMA,
            pltpu.SemaphoreType.DMA,
        ],
    )(table, idx)


_T1_RB = 2000


def _t1_body(w_ref, qW_ref, qb_ref, dW_ref, db_ref, out_ref):
    i = pl.program_id(0)
    x = w_ref[...]
    yq = jnp.tanh(jnp.dot(x, qW_ref[...], preferred_element_type=jnp.float32)
                  + qb_ref[0][None, :])
    yd = jnp.tanh(jnp.dot(x, dW_ref[...], preferred_element_type=jnp.float32)
                  + db_ref[0][None, :])
    y = jnp.concatenate([yq, yd], axis=1)
    row = lax.broadcasted_iota(jnp.int32, (_T1_RB, 1), 0) + i * _T1_RB
    out_ref[...] = jnp.where(row == 1, 0.0, y)


def _t1(word_emb, query_W, query_b, key_doc_W, key_doc_b):
    nblk = NWRD // _T1_RB
    return pl.pallas_call(
        _t1_body,
        grid=(nblk,),
        in_specs=[
            pl.BlockSpec((_T1_RB, D), lambda i: (i, 0)),
            pl.BlockSpec((D, D), lambda i: (0, 0)),
            pl.BlockSpec((1, D), lambda i: (0, 0)),
            pl.BlockSpec((D, D), lambda i: (0, 0)),
            pl.BlockSpec((1, D), lambda i: (0, 0)),
        ],
        out_specs=pl.BlockSpec((_T1_RB, 2 * D), lambda i: (i, 0)),
        out_shape=jax.ShapeDtypeStruct((NWRD, 2 * D), jnp.float32),
    )(word_emb, query_W, query_b[None, :], key_doc_W, key_doc_b[None, :])


def _t2_body(x_ref, par_ref, ids_ref, attW_ref, attv_ref, relW_ref, relb_ref,
             kkbW_ref, kkbb_ref, out_ref):
    xs = []
    cols = []
    for l in range(RL):
        x2 = x_ref[:, l, :]
        parl = par_ref[:, l][:, None]
        xl = jnp.where(parl == 1, x2[:, D:], x2[:, :D])
        xs.append(xl)
        tl = jnp.tanh(jnp.dot(xl, attW_ref[...], preferred_element_type=jnp.float32))
        cols.append(jnp.dot(tl, attv_ref[...], preferred_element_type=jnp.float32))
    s = jnp.concatenate(cols, axis=1)
    mask = ids_ref[...] != 1
    s = jnp.where(mask, s, -1e9)
    mx = jnp.max(s, axis=1, keepdims=True)
    e = jnp.exp(s - mx)
    a = e / jnp.sum(e, axis=1, keepdims=True)
    agg = jnp.zeros((NR, D), jnp.float32)
    for l in range(RL):
        agg = agg + a[:, l][:, None] * xs[l]
    rel_enc = jnp.tanh(jnp.dot(agg, relW_ref[...], preferred_element_type=jnp.float32)
                       + relb_ref[0][None, :])
    t2 = jnp.tanh(jnp.dot(rel_enc, kkbW_ref[...], preferred_element_type=jnp.float32)
                  + kkbb_ref[0][None, :])
    row = lax.broadcasted_iota(jnp.int32, (NR, 1), 0)
    t2 = jnp.where(row == 0, 0.0, t2)
    out_ref[...] = jnp.concatenate([t2, jnp.zeros((NR, D), jnp.float32)], axis=1)


def _t2(rel_rows, rel_par, rel_word_ids, att_W, att_v, rel_W, rel_b,
        key_kb_W, key_kb_b):
    return pl.pallas_call(
        _t2_body,
        out_shape=jax.ShapeDtypeStruct((NR, 2 * D), jnp.float32),
    )(rel_rows, rel_par, rel_word_ids, att_W, att_v[:, None], rel_W,
      rel_b[None, :], key_kb_W, key_kb_b[None, :])


_T3_RB = 6400


def _lrelu(x):
    return jnp.where(x >= 0, x, 0.01 * x)


def _psel(rows2, par):
    return jnp.where(par == 1, rows2[:, D:], rows2[:, :D])


def _t3a_body(sub_ref, rel_ref, idx_ref, par_ref, eW_ref, eb_ref, kW_ref,
              kb_ref, out_ref):
    x = _psel(sub_ref[...], par_ref[...])
    h = _lrelu(jnp.dot(x, eW_ref[...], preferred_element_type=jnp.float32)
               + eb_ref[0][None, :])
    h = jnp.tanh(jnp.dot(h, kW_ref[...], preferred_element_type=jnp.float32)
                 + kb_ref[0][None, :])
    mask = idx_ref[...] != 0
    out_ref[...] = jnp.where(mask, h, 0.0) + rel_ref[:, :D]


def _t3a(ent_rows, rel_rows, kk_sub_flat, par_flat, eW, eb, kW, kb):
    grid = (B * M) // _T3_RB
    bs = lambda i: (i, 0)
    return pl.pallas_call(
        _t3a_body,
        grid=(grid,),
        in_specs=[
            pl.BlockSpec((_T3_RB, 2 * D), bs),
            pl.BlockSpec((_T3_RB, 2 * D), bs),
            pl.BlockSpec((_T3_RB, 1), bs),
            pl.BlockSpec((_T3_RB, 1), bs),
            pl.BlockSpec((D, D), lambda i: (0, 0)),
            pl.BlockSpec((1, D), lambda i: (0, 0)),
            pl.BlockSpec((D, D), lambda i: (0, 0)),
            pl.BlockSpec((1, D), lambda i: (0, 0)),
        ],
        out_specs=pl.BlockSpec((_T3_RB, D), bs),
        out_shape=jax.ShapeDtypeStruct((B * M, D), jnp.float32),
    )(ent_rows, rel_rows, kk_sub_flat, par_flat, eW, eb[None, :], kW,
      kb[None, :])


def _t3b_body(v_ref, idx_ref, par_ref, W_ref, b_ref, out_ref):
    x = _psel(v_ref[...], par_ref[...])
    h = jnp.tanh(jnp.dot(x, W_ref[...], preferred_element_type=jnp.float32)
                 + b_ref[0][None, :])
    mask = idx_ref[...] != 0
    out_ref[...] = jnp.where(mask, h, 0.0)


def _t3b(ent_rows, off_blk, n, vidx_flat, par_flat, W, b):
    grid = n // _T3_RB
    return pl.pallas_call(
        _t3b_body,
        grid=(grid,),
        in_specs=[
            pl.BlockSpec((_T3_RB, 2 * D), lambda i: (i + off_blk, 0)),
            pl.BlockSpec((_T3_RB, 1), lambda i: (i, 0)),
            pl.BlockSpec((_T3_RB, 1), lambda i: (i, 0)),
            pl.BlockSpec((D, D), lambda i: (0, 0)),
            pl.BlockSpec((1, D), lambda i: (0, 0)),
        ],
        out_specs=pl.BlockSpec((_T3_RB, D), lambda i: (i, 0)),
        out_shape=jax.ShapeDtypeStruct((n, D), jnp.float32),
    )(ent_rows, vidx_flat, par_flat, W, b[None, :])


def _t3c_body(c_ref, par_ref, W_ref, b_ref, out_ref):
    x = _psel(c_ref[...], par_ref[...])
    out_ref[...] = _lrelu(jnp.dot(x, W_ref[...],
                                  preferred_element_type=jnp.float32)
                          + b_ref[0][None, :])


def _t3c(ent_rows, off_blk, n, par_flat, W, b):
    grid = n // _T3_RB
    return pl.pallas_call(
        _t3c_body,
        grid=(grid,),
        in_specs=[
            pl.BlockSpec((_T3_RB, 2 * D), lambda i: (i + off_blk, 0)),
            pl.BlockSpec((_T3_RB, 1), lambda i: (i, 0)),
            pl.BlockSpec((D, D), lambda i: (0, 0)),
            pl.BlockSpec((1, D), lambda i: (0, 0)),
        ],
        out_specs=pl.BlockSpec((_T3_RB, D), lambda i: (i, 0)),
        out_shape=jax.ShapeDtypeStruct((n, D), jnp.float32),
    )(ent_rows, par_flat, W, b[None, :])


_T4_BB = 64


def _t4_body(ekkb_ref, ekd_ref, evk_ref, evd_ref, ecand_ref, q_ref, ans_ref,
             oW_ref, ob_ref, lsum_ref, pred_ref, dist_ref):
    i = pl.program_id(0)
    q = q_ref[...]
    ekkb = ekkb_ref[...]
    ekd = ekd_ref[...]
    evk = evk_ref[...]
    evd = evd_ref[...]
    for _ in range(NUM_HOP):
        ph_kb = jnp.sum(ekkb * q[:, None, :], axis=2)
        ph_kd = jnp.sum(ekd * q[:, None, :], axis=2)
        ph_kb = jnp.where(ph_kb == 0.0, -1e9, ph_kb)
        ph_kd = jnp.where(ph_kd == 0.0, -1e9, ph_kd)
        mx = jnp.maximum(jnp.max(ph_kb, axis=1, keepdims=True),
                         jnp.max(ph_kd, axis=1, keepdims=True))
        e_kb = jnp.exp(ph_kb - mx)
        e_kd = jnp.exp(ph_kd - mx)
        tot = (jnp.sum(e_kb, axis=1, keepdims=True)
               + jnp.sum(e_kd, axis=1, keepdims=True))
        sc_kb = e_kb / tot
        sc_kd = e_kd / tot
        out = (jnp.sum(sc_kb[:, :, None] * evk, axis=1)
               + jnp.sum(sc_kd[:, :, None] * evd, axis=1))
        q = (jnp.dot(q + out, oW_ref[...], preferred_element_type=jnp.float32)
             + ob_ref[0][None, :])
    sp = jnp.sum(ecand_ref[...] * q[:, None, :], axis=2)
    y = ans_ref[...].astype(jnp.float32)
    contrib = jnp.sum(jnp.maximum(sp, 0.0) - sp * y
                      + jnp.log1p(jnp.exp(-jnp.abs(sp))))
    prev = jnp.where(i == 0, 0.0, lsum_ref[0, 0])
    lsum_ref[...] = jnp.full((1, 1), prev + contrib, dtype=jnp.float32)
    pred_ref[...] = jnp.argmax(sp, axis=1).astype(jnp.int32)[:, None]
    dist_ref[...] = 1.0 / (1.0 + jnp.exp(-sp))


def _t4(ekkb3, ekd3, evk3, evd3, ecand3, q0, answers, out_W, out_b):
    grid = B // _T4_BB
    return pl.pallas_call(
        _t4_body,
        grid=(grid,),
        in_specs=[
            pl.BlockSpec((_T4_BB, M, D), lambda i: (i, 0, 0)),
            pl.BlockSpec((_T4_BB, MD, D), lambda i: (i, 0, 0)),
            pl.BlockSpec((_T4_BB, M, D), lambda i: (i, 0, 0)),
            pl.BlockSpec((_T4_BB, MD, D), lambda i: (i, 0, 0)),
            pl.BlockSpec((_T4_BB, C, D), lambda i: (i, 0, 0)),
            pl.BlockSpec((_T4_BB, D), lambda i: (i, 0)),
            pl.BlockSpec((_T4_BB, C), lambda i: (i, 0)),
            pl.BlockSpec((D, D), lambda i: (0, 0)),
            pl.BlockSpec((1, D), lambda i: (0, 0)),
        ],
        out_specs=(
            pl.BlockSpec((1, 1), lambda i: (0, 0)),
            pl.BlockSpec((_T4_BB, 1), lambda i: (i, 0)),
            pl.BlockSpec((_T4_BB, C), lambda i: (i, 0)),
        ),
        out_shape=(
            jax.ShapeDtypeStruct((1, 1), jnp.float32),
            jax.ShapeDtypeStruct((B, 1), jnp.int32),
            jax.ShapeDtypeStruct((B, C), jnp.float32),
        ),
    )(ekkb3, ekd3, evk3, evd3, ecand3, q0, answers, out_W, out_b[None, :])


def kernel(questions, key_kb, rel_word_ids, key_doc, val_kb, val_doc,
           candidate_entities, answers, entity_emb, word_emb,
           entity_linear_W, entity_linear_b, att_W, att_v,
           rel_W, rel_b, query_W, query_b, key_kb_W, key_kb_b,
           key_doc_W, key_doc_b, value_W, value_b, out_W, out_b):
    i32 = jnp.int32
    kk_sub = key_kb[:, :, 0].reshape(-1).astype(i32)
    kk_rel = key_kb[:, :, 1].reshape(-1).astype(i32)
    vk = val_kb.reshape(-1).astype(i32)
    vd = val_doc.reshape(-1).astype(i32)
    cand = candidate_entities.reshape(-1).astype(i32)

    ent_pairs = entity_emb[:NENT].reshape(NENT // 2, 2 * D)
    wrd_pairs = word_emb.reshape(NWRD // 2, 2 * D)

    eidx = jnp.concatenate([
        kk_sub, vk, vd, cand,
        jnp.zeros((_A_G1 - 460800,), i32),
    ])
    widx = jnp.concatenate([
        rel_word_ids.reshape(-1).astype(i32),
        jnp.zeros((_A_G2 - NR * RL,), i32),
    ])
    ent_rows, wrd_rows = _sc_a(ent_pairs, wrd_pairs,
                               eidx >> 1, widx >> 1)

    word_tables = _t1(word_emb, query_W, query_b, key_doc_W, key_doc_b)

    rel_par = (rel_word_ids & 1).astype(i32)
    rel_table2 = _t2(wrd_rows[:NR * RL].reshape(NR, RL, 2 * D), rel_par,
                     rel_word_ids, att_W, att_v, rel_W, rel_b,
                     key_kb_W, key_kb_b)

    bidx = jnp.concatenate([
        questions.reshape(-1).astype(i32),
        key_doc.reshape(-1).astype(i32),
        jnp.zeros(((_B_G - B - B * MD) * _B_S,), i32),
    ])
    bsum = _sc_b(word_tables, bidx)
    q0 = bsum[:B, :D]
    ekd = bsum[B:B + B * MD, D:]

    cidx = jnp.concatenate([
        kk_rel, jnp.zeros((_C_G - B * M,), i32),
    ])
    rel_slot_rows = _sc_c(rel_table2, cidx)

    ekkb = _t3a(ent_rows, rel_slot_rows, kk_sub[:, None],
                (kk_sub & 1)[:, None], entity_linear_W, entity_linear_b,
                key_kb_W, key_kb_b)
    vall = jnp.concatenate([vk, vd])
    ev = _t3b(ent_rows, 102400 // _T3_RB, 153600, vall[:, None],
              (vall & 1)[:, None], value_W, value_b)
    ecand = _t3c(ent_rows, 256000 // _T3_RB, 204800,
                 (cand & 1)[:, None], entity_linear_W, entity_linear_b)

    lsum, pred, pred_dist = _t4(
        ekkb.reshape(B, M, D), ekd.reshape(B, MD, D),
        ev[:B * M].reshape(B, M, D), ev[B * M:].reshape(B, MD, D),
        ecand.reshape(B, C, D), q0, answers.astype(i32), out_W, out_b)
    loss = lsum[0, 0] / (B * C)
    return (loss, pred[:, 0], pred_dist)


# speedup vs baseline: 1.2828x; 1.1445x over previous
import jax
import jax.numpy as jnp
from jax import lax
from jax.experimental import pallas as pl
from jax.experimental.pallas import tpu as pltpu
from jax.experimental.pallas import tpu_sc as plsc

B = 1024
QL = 20
M = 100
MD = 50
DL = 20
RL = 10
NR = 1000
NENT = 1000000
NWRD = 100000
D = 64
C = 200
NUM_HOP = 3

NC = 2
NS = 16
NWK = NC * NS

_MESH = plsc.VectorSubcoreMesh(core_axis_name="c", subcore_axis_name="s")


def _wid():
    return lax.axis_index("s") * NC + lax.axis_index("c")


_A_G1 = 491520
_A_PW1 = _A_G1 // NWK
_A_R = 384
_A_NCH = _A_PW1 // _A_R
_A_G2 = 12288
_A_PW2 = _A_G2 // NWK


def _sc_a_body(ent_hbm, wrd_hbm, eidx_hbm, widx_hbm, ent_out, wrd_out,
               idx0, idx1, rows0, rows1, sem0, sem1):
    wid = _wid()
    bufs = ((idx0, rows0, sem0), (idx1, rows1, sem1))

    def fire(c, par):
        idxb, rowsb, semb = bufs[par]
        pltpu.sync_copy(eidx_hbm.at[pl.ds(wid * _A_PW1 + c * _A_R, _A_R)], idxb)
        for k in range(3):
            pltpu.async_copy(ent_hbm.at[idxb.at[pl.ds(k * 128, 128)]],
                             rowsb.at[pl.ds(k * 128, 128)], semb)

    def drain(par):
        idxb, rowsb, semb = bufs[par]
        for k in range(3):
            pltpu.make_async_copy(ent_hbm.at[idxb.at[pl.ds(k * 128, 128)]],
                                  rowsb.at[pl.ds(k * 128, 128)], semb).wait()

    fire(0, 0)
    fire(1, 1)

    @pl.loop(0, _A_NCH, step=2)
    def _(c0):
        for par in range(2):
            c = c0 + par
            idxb, rowsb, semb = bufs[par]
            drain(par)
            pltpu.sync_copy(rowsb, ent_out.at[pl.ds(wid * _A_PW1 + c * _A_R, _A_R)])
            nxt = c + 2

            @pl.when(nxt < _A_NCH)
            def _():
                fire(nxt, par)

    for k in range(3):
        pltpu.sync_copy(widx_hbm.at[pl.ds(wid * _A_PW2 + k * 128, 128)],
                        idx0.at[pl.ds(0, 128)])
        pltpu.async_copy(wrd_hbm.at[idx0.at[pl.ds(0, 128)]],
                         rows0.at[pl.ds(0, 128)], sem0)
        pltpu.make_async_copy(wrd_hbm.at[idx0.at[pl.ds(0, 128)]],
                              rows0.at[pl.ds(0, 128)], sem0).wait()
        pltpu.sync_copy(rows0.at[pl.ds(0, 128)],
                        wrd_out.at[pl.ds(wid * _A_PW2 + k * 128, 128)])


def _sc_a(ent_pairs, wrd_pairs, eidx, widx):
    return pl.kernel(
        _sc_a_body,
        out_type=(jax.ShapeDtypeStruct((_A_G1, 2 * D), jnp.float32),
                  jax.ShapeDtypeStruct((_A_G2, 2 * D), jnp.float32)),
        mesh=_MESH,
        scratch_types=[
            pltpu.VMEM((_A_R,), jnp.int32),
            pltpu.VMEM((_A_R,), jnp.int32),
            pltpu.VMEM((_A_R, 2 * D), jnp.float32),
            pltpu.VMEM((_A_R, 2 * D), jnp.float32),
            pltpu.SemaphoreType.DMA,
            pltpu.SemaphoreType.DMA,
        ],
    )(ent_pairs, wrd_pairs, eidx, widx)


_B_S = 20
_B_G = 53248
_B_PW = _B_G // NWK
_B_CH = 16
_B_R = _B_CH * _B_S
_B_NCH = _B_PW // _B_CH


def _sc_b_body(tab_hbm, idx_hbm, out_hbm,
               idx0, idx1, rows0, rows1, outv, sem0, sem1):
    wid = _wid()
    bufs = ((idx0, rows0, sem0), (idx1, rows1, sem1))
    npw = _B_PW * _B_S
    slc = ((0, 128), (128, 128), (256, 64))

    def fire(c, par):
        idxb, rowsb, semb = bufs[par]
        pltpu.sync_copy(idx_hbm.at[pl.ds(wid * npw + c * _B_R, _B_R)], idxb)
        for o, n in slc:
            pltpu.async_copy(tab_hbm.at[idxb.at[pl.ds(o, n)]],
                             rowsb.at[pl.ds(o, n)], semb)

    def drain(par):
        idxb, rowsb, semb = bufs[par]
        for o, n in slc:
            pltpu.make_async_copy(tab_hbm.at[idxb.at[pl.ds(o, n)]],
                                  rowsb.at[pl.ds(o, n)], semb).wait()

    fire(0, 0)
    fire(1, 1)

    @pl.loop(0, _B_NCH, step=2)
    def _(c0):
        for par in range(2):
            c = c0 + par
            idxb, rowsb, semb = bufs[par]
            drain(par)

            @pl.loop(0, _B_CH)
            def _(g):
                r0 = g * _B_S
                for dd in range(8):
                    sl = pl.ds(dd * 16, 16)
                    acc = rowsb[r0, sl]
                    for s in range(1, _B_S):
                        acc = acc + rowsb[r0 + s, sl]
                    outv[g, sl] = acc

            pltpu.sync_copy(outv, out_hbm.at[pl.ds(wid * _B_PW + c * _B_CH, _B_CH)])
            nxt = c + 2

            @pl.when(nxt < _B_NCH)
            def _():
                fire(nxt, par)


def _sc_b(table, idx):
    return pl.kernel(
        _sc_b_body,
        out_type=jax.ShapeDtypeStruct((_B_G, 2 * D), jnp.float32),
        mesh=_MESH,
        scratch_types=[
            pltpu.VMEM((_B_R,), jnp.int32),
            pltpu.VMEM((_B_R,), jnp.int32),
            pltpu.VMEM((_B_R, 2 * D), jnp.float32),
            pltpu.VMEM((_B_R, 2 * D), jnp.float32),
            pltpu.VMEM((_B_CH, 2 * D), jnp.float32),
            pltpu.SemaphoreType.DMA,
            pltpu.SemaphoreType.DMA,
        ],
    )(table, idx)


_T1_RB = 2000


def _t1_body(w_ref, qW_ref, qb_ref, dW_ref, db_ref, out_ref):
    i = pl.program_id(0)
    x = w_ref[...]
    yq = jnp.tanh(jnp.dot(x, qW_ref[...], preferred_element_type=jnp.float32)
                  + qb_ref[0][None, :])
    yd = jnp.tanh(jnp.dot(x, dW_ref[...], preferred_element_type=jnp.float32)
                  + db_ref[0][None, :])
    y = jnp.concatenate([yq, yd], axis=1)
    row = lax.broadcasted_iota(jnp.int32, (_T1_RB, 1), 0) + i * _T1_RB
    out_ref[...] = jnp.where(row == 1, 0.0, y)


def _t1(word_emb, query_W, query_b, key_doc_W, key_doc_b):
    nblk = NWRD // _T1_RB
    return pl.pallas_call(
        _t1_body,
        grid=(nblk,),
        in_specs=[
            pl.BlockSpec((_T1_RB, D), lambda i: (i, 0)),
            pl.BlockSpec((D, D), lambda i: (0, 0)),
            pl.BlockSpec((1, D), lambda i: (0, 0)),
            pl.BlockSpec((D, D), lambda i: (0, 0)),
            pl.BlockSpec((1, D), lambda i: (0, 0)),
        ],
        out_specs=pl.BlockSpec((_T1_RB, 2 * D), lambda i: (i, 0)),
        out_shape=jax.ShapeDtypeStruct((NWRD, 2 * D), jnp.float32),
    )(word_emb, query_W, query_b[None, :], key_doc_W, key_doc_b[None, :])


def _t2_body(x_ref, par_ref, ids_ref, attW_ref, attv_ref, relW_ref, relb_ref,
             kkbW_ref, kkbb_ref, out_ref):
    xs = []
    cols = []
    for l in range(RL):
        x2 = x_ref[:, l, :]
        parl = par_ref[:, l][:, None]
        xl = jnp.where(parl == 1, x2[:, D:], x2[:, :D])
        xs.append(xl)
        tl = jnp.tanh(jnp.dot(xl, attW_ref[...], preferred_element_type=jnp.float32))
        cols.append(jnp.dot(tl, attv_ref[...], preferred_element_type=jnp.float32))
    s = jnp.concatenate(cols, axis=1)
    mask = ids_ref[...] != 1
    s = jnp.where(mask, s, -1e9)
    mx = jnp.max(s, axis=1, keepdims=True)
    e = jnp.exp(s - mx)
    a = e / jnp.sum(e, axis=1, keepdims=True)
    agg = jnp.zeros((NR, D), jnp.float32)
    for l in range(RL):
        agg = agg + a[:, l][:, None] * xs[l]
    rel_enc = jnp.tanh(jnp.dot(agg, relW_ref[...], preferred_element_type=jnp.float32)
                       + relb_ref[0][None, :])
    t2 = jnp.tanh(jnp.dot(rel_enc, kkbW_ref[...], preferred_element_type=jnp.float32)
                  + kkbb_ref[0][None, :])
    row = lax.broadcasted_iota(jnp.int32, (NR, 1), 0)
    out_ref[...] = jnp.where(row == 0, 0.0, t2)


def _t2(rel_rows, rel_par, rel_word_ids, att_W, att_v, rel_W, rel_b,
        key_kb_W, key_kb_b):
    return pl.pallas_call(
        _t2_body,
        out_shape=jax.ShapeDtypeStruct((NR, D), jnp.float32),
    )(rel_rows, rel_par, rel_word_ids, att_W, att_v[:, None], rel_W,
      rel_b[None, :], key_kb_W, key_kb_b[None, :])


_T3_RB = 6400


def _lrelu(x):
    return jnp.where(x >= 0, x, 0.01 * x)


def _psel(rows2, par):
    return jnp.where(par == 1, rows2[:, D:], rows2[:, :D])


_T3A_RB = 2048


def _t3a_body(sub_ref, tab_ref, idx_ref, par_ref, rel_ref, eW_ref, eb_ref,
              kW_ref, kb_ref, out_ref):
    x = _psel(sub_ref[...], par_ref[...])
    h = _lrelu(jnp.dot(x, eW_ref[...], preferred_element_type=jnp.float32)
               + eb_ref[0][None, :])
    h = jnp.tanh(jnp.dot(h, kW_ref[...], preferred_element_type=jnp.float32)
                 + kb_ref[0][None, :])
    mask = idx_ref[...] != 0
    col = lax.broadcasted_iota(jnp.int32, (_T3A_RB, NR), 1)
    oh = (col == rel_ref[...]).astype(jnp.float32)
    rel = jnp.dot(oh, tab_ref[...], preferred_element_type=jnp.float32)
    out_ref[...] = jnp.where(mask, h, 0.0) + rel


def _t3a(ent_rows, rel_table2, kk_sub_flat, par_flat, kk_rel_flat,
         eW, eb, kW, kb):
    grid = (B * M) // _T3A_RB
    bs = lambda i: (i, 0)
    return pl.pallas_call(
        _t3a_body,
        grid=(grid,),
        in_specs=[
            pl.BlockSpec((_T3A_RB, 2 * D), bs),
            pl.BlockSpec((NR, D), lambda i: (0, 0)),
            pl.BlockSpec((_T3A_RB, 1), bs),
            pl.BlockSpec((_T3A_RB, 1), bs),
            pl.BlockSpec((_T3A_RB, 1), bs),
            pl.BlockSpec((D, D), lambda i: (0, 0)),
            pl.BlockSpec((1, D), lambda i: (0, 0)),
            pl.BlockSpec((D, D), lambda i: (0, 0)),
            pl.BlockSpec((1, D), lambda i: (0, 0)),
        ],
        out_specs=pl.BlockSpec((_T3A_RB, D), bs),
        out_shape=jax.ShapeDtypeStruct((B * M, D), jnp.float32),
    )(ent_rows, rel_table2, kk_sub_flat, par_flat, kk_rel_flat, eW,
      eb[None, :], kW, kb[None, :])


def _t3b_body(v_ref, idx_ref, par_ref, W_ref, b_ref, out_ref):
    x = _psel(v_ref[...], par_ref[...])
    h = jnp.tanh(jnp.dot(x, W_ref[...], preferred_element_type=jnp.float32)
                 + b_ref[0][None, :])
    mask = idx_ref[...] != 0
    out_ref[...] = jnp.where(mask, h, 0.0)


def _t3b(ent_rows, off_blk, n, vidx_flat, par_flat, W, b):
    grid = n // _T3_RB
    return pl.pallas_call(
        _t3b_body,
        grid=(grid,),
        in_specs=[
            pl.BlockSpec((_T3_RB, 2 * D), lambda i: (i + off_blk, 0)),
            pl.BlockSpec((_T3_RB, 1), lambda i: (i, 0)),
            pl.BlockSpec((_T3_RB, 1), lambda i: (i, 0)),
            pl.BlockSpec((D, D), lambda i: (0, 0)),
            pl.BlockSpec((1, D), lambda i: (0, 0)),
        ],
        out_specs=pl.BlockSpec((_T3_RB, D), lambda i: (i, 0)),
        out_shape=jax.ShapeDtypeStruct((n, D), jnp.float32),
    )(ent_rows, vidx_flat, par_flat, W, b[None, :])


def _t3c_body(c_ref, par_ref, W_ref, b_ref, out_ref):
    x = _psel(c_ref[...], par_ref[...])
    out_ref[...] = _lrelu(jnp.dot(x, W_ref[...],
                                  preferred_element_type=jnp.float32)
                          + b_ref[0][None, :])


def _t3c(ent_rows, off_blk, n, par_flat, W, b):
    grid = n // _T3_RB
    return pl.pallas_call(
        _t3c_body,
        grid=(grid,),
        in_specs=[
            pl.BlockSpec((_T3_RB, 2 * D), lambda i: (i + off_blk, 0)),
            pl.BlockSpec((_T3_RB, 1), lambda i: (i, 0)),
            pl.BlockSpec((D, D), lambda i: (0, 0)),
            pl.BlockSpec((1, D), lambda i: (0, 0)),
        ],
        out_specs=pl.BlockSpec((_T3_RB, D), lambda i: (i, 0)),
        out_shape=jax.ShapeDtypeStruct((n, D), jnp.float32),
    )(ent_rows, par_flat, W, b[None, :])


_T4_BB = 64


def _t4_body(ekkb_ref, ekd_ref, evk_ref, evd_ref, ecand_ref, q_ref, ans_ref,
             oW_ref, ob_ref, lsum_ref, pred_ref, dist_ref):
    i = pl.program_id(0)
    q = q_ref[...]
    ekkb = ekkb_ref[...]
    ekd = ekd_ref[...]
    evk = evk_ref[...]
    evd = evd_ref[...]
    for _ in range(NUM_HOP):
        ph_kb = jnp.sum(ekkb * q[:, None, :], axis=2)
        ph_kd = jnp.sum(ekd * q[:, None, :], axis=2)
        ph_kb = jnp.where(ph_kb == 0.0, -1e9, ph_kb)
        ph_kd = jnp.where(ph_kd == 0.0, -1e9, ph_kd)
        mx = jnp.maximum(jnp.max(ph_kb, axis=1, keepdims=True),
                         jnp.max(ph_kd, axis=1, keepdims=True))
        e_kb = jnp.exp(ph_kb - mx)
        e_kd = jnp.exp(ph_kd - mx)
        tot = (jnp.sum(e_kb, axis=1, keepdims=True)
               + jnp.sum(e_kd, axis=1, keepdims=True))
        sc_kb = e_kb / tot
        sc_kd = e_kd / tot
        out = (jnp.sum(sc_kb[:, :, None] * evk, axis=1)
               + jnp.sum(sc_kd[:, :, None] * evd, axis=1))
        q = (jnp.dot(q + out, oW_ref[...], preferred_element_type=jnp.float32)
             + ob_ref[0][None, :])
    sp = jnp.sum(ecand_ref[...] * q[:, None, :], axis=2)
    y = ans_ref[...].astype(jnp.float32)
    contrib = jnp.sum(jnp.maximum(sp, 0.0) - sp * y
                      + jnp.log1p(jnp.exp(-jnp.abs(sp))))
    prev = jnp.where(i == 0, 0.0, lsum_ref[0, 0])
    lsum_ref[...] = jnp.full((1, 1), prev + contrib, dtype=jnp.float32)
    pred_ref[...] = jnp.argmax(sp, axis=1).astype(jnp.int32)[:, None]
    dist_ref[...] = 1.0 / (1.0 + jnp.exp(-sp))


def _t4(ekkb3, ekd3, evk3, evd3, ecand3, q0, answers, out_W, out_b):
    grid = B // _T4_BB
    return pl.pallas_call(
        _t4_body,
        grid=(grid,),
        in_specs=[
            pl.BlockSpec((_T4_BB, M, D), lambda i: (i, 0, 0)),
            pl.BlockSpec((_T4_BB, MD, D), lambda i: (i, 0, 0)),
            pl.BlockSpec((_T4_BB, M, D), lambda i: (i, 0, 0)),
            pl.BlockSpec((_T4_BB, MD, D), lambda i: (i, 0, 0)),
            pl.BlockSpec((_T4_BB, C, D), lambda i: (i, 0, 0)),
            pl.BlockSpec((_T4_BB, D), lambda i: (i, 0)),
            pl.BlockSpec((_T4_BB, C), lambda i: (i, 0)),
            pl.BlockSpec((D, D), lambda i: (0, 0)),
            pl.BlockSpec((1, D), lambda i: (0, 0)),
        ],
        out_specs=(
            pl.BlockSpec((1, 1), lambda i: (0, 0)),
            pl.BlockSpec((_T4_BB, 1), lambda i: (i, 0)),
            pl.BlockSpec((_T4_BB, C), lambda i: (i, 0)),
        ),
        out_shape=(
            jax.ShapeDtypeStruct((1, 1), jnp.float32),
            jax.ShapeDtypeStruct((B, 1), jnp.int32),
            jax.ShapeDtypeStruct((B, C), jnp.float32),
        ),
    )(ekkb3, ekd3, evk3, evd3, ecand3, q0, answers, out_W, out_b[None, :])


def kernel(questions, key_kb, rel_word_ids, key_doc, val_kb, val_doc,
           candidate_entities, answers, entity_emb, word_emb,
           entity_linear_W, entity_linear_b, att_W, att_v,
           rel_W, rel_b, query_W, query_b, key_kb_W, key_kb_b,
           key_doc_W, key_doc_b, value_W, value_b, out_W, out_b):
    i32 = jnp.int32
    kk_sub = key_kb[:, :, 0].reshape(-1).astype(i32)
    kk_rel = key_kb[:, :, 1].reshape(-1).astype(i32)
    vk = val_kb.reshape(-1).astype(i32)
    vd = val_doc.reshape(-1).astype(i32)
    cand = candidate_entities.reshape(-1).astype(i32)

    ent_pairs = entity_emb[:NENT].reshape(NENT // 2, 2 * D)
    wrd_pairs = word_emb.reshape(NWRD // 2, 2 * D)

    eidx = jnp.concatenate([
        kk_sub, vk, vd, cand,
        jnp.zeros((_A_G1 - 460800,), i32),
    ])
    widx = jnp.concatenate([
        rel_word_ids.reshape(-1).astype(i32),
        jnp.zeros((_A_G2 - NR * RL,), i32),
    ])
    ent_rows, wrd_rows = _sc_a(ent_pairs, wrd_pairs,
                               eidx >> 1, widx >> 1)

    word_tables = _t1(word_emb, query_W, query_b, key_doc_W, key_doc_b)

    rel_par = (rel_word_ids & 1).astype(i32)
    rel_table2 = _t2(wrd_rows[:NR * RL].reshape(NR, RL, 2 * D), rel_par,
                     rel_word_ids, att_W, att_v, rel_W, rel_b,
                     key_kb_W, key_kb_b)

    bidx = jnp.concatenate([
        questions.reshape(-1).astype(i32),
        key_doc.reshape(-1).astype(i32),
        jnp.zeros(((_B_G - B - B * MD) * _B_S,), i32),
    ])
    bsum = _sc_b(word_tables, bidx)
    q0 = bsum[:B, :D]
    ekd = bsum[B:B + B * MD, D:]

    ekkb = _t3a(ent_rows, rel_table2, kk_sub[:, None],
                (kk_sub & 1)[:, None], kk_rel[:, None],
                entity_linear_W, entity_linear_b, key_kb_W, key_kb_b)
    vall = jnp.concatenate([vk, vd])
    ev = _t3b(ent_rows, 102400 // _T3_RB, 153600, vall[:, None],
              (vall & 1)[:, None], value_W, value_b)
    ecand = _t3c(ent_rows, 256000 // _T3_RB, 204800,
                 (cand & 1)[:, None], entity_linear_W, entity_linear_b)

    lsum, pred, pred_dist = _t4(
        ekkb.reshape(B, M, D), ekd.reshape(B, MD, D),
        ev[:B * M].reshape(B, M, D), ev[B * M:].reshape(B, MD, D),
        ecand.reshape(B, C, D), q0, answers.astype(i32), out_W, out_b)
    loss = lsum[0, 0] / (B * C)
    return (loss, pred[:, 0], pred_dist)


# speedup vs baseline: 1.2940x; 1.0087x over previous
import jax
import jax.numpy as jnp
from jax import lax
from jax.experimental import pallas as pl
from jax.experimental.pallas import tpu as pltpu
from jax.experimental.pallas import tpu_sc as plsc

B = 1024
QL = 20
M = 100
MD = 50
DL = 20
RL = 10
NR = 1000
NENT = 1000000
NWRD = 100000
D = 64
C = 200
NUM_HOP = 3

NC = 2
NS = 16
NWK = NC * NS

_MESH = plsc.VectorSubcoreMesh(core_axis_name="c", subcore_axis_name="s")


def _wid():
    return lax.axis_index("s") * NC + lax.axis_index("c")


_A_G1 = 491520
_A_PW1 = _A_G1 // NWK
_A_R = 384
_A_NCH = _A_PW1 // _A_R
_A_G2 = 12288
_A_PW2 = _A_G2 // NWK


def _sc_a_body(ent_hbm, wrd_hbm, eidx_hbm, widx_hbm, ent_out, wrd_out,
               idx0, idx1, rows0, rows1, sem0, sem1):
    wid = _wid()
    bufs = ((idx0, rows0, sem0), (idx1, rows1, sem1))

    def fire(c, par):
        idxb, rowsb, semb = bufs[par]
        pltpu.sync_copy(eidx_hbm.at[pl.ds(wid * _A_PW1 + c * _A_R, _A_R)], idxb)
        for k in range(3):
            pltpu.async_copy(ent_hbm.at[idxb.at[pl.ds(k * 128, 128)]],
                             rowsb.at[pl.ds(k * 128, 128)], semb)

    def drain(par):
        idxb, rowsb, semb = bufs[par]
        for k in range(3):
            pltpu.make_async_copy(ent_hbm.at[idxb.at[pl.ds(k * 128, 128)]],
                                  rowsb.at[pl.ds(k * 128, 128)], semb).wait()

    fire(0, 0)
    fire(1, 1)

    @pl.loop(0, _A_NCH, step=2)
    def _(c0):
        for par in range(2):
            c = c0 + par
            idxb, rowsb, semb = bufs[par]
            drain(par)
            pltpu.sync_copy(rowsb, ent_out.at[pl.ds(wid * _A_PW1 + c * _A_R, _A_R)])
            nxt = c + 2

            @pl.when(nxt < _A_NCH)
            def _():
                fire(nxt, par)

    for k in range(3):
        pltpu.sync_copy(widx_hbm.at[pl.ds(wid * _A_PW2 + k * 128, 128)],
                        idx0.at[pl.ds(0, 128)])
        pltpu.async_copy(wrd_hbm.at[idx0.at[pl.ds(0, 128)]],
                         rows0.at[pl.ds(0, 128)], sem0)
        pltpu.make_async_copy(wrd_hbm.at[idx0.at[pl.ds(0, 128)]],
                              rows0.at[pl.ds(0, 128)], sem0).wait()
        pltpu.sync_copy(rows0.at[pl.ds(0, 128)],
                        wrd_out.at[pl.ds(wid * _A_PW2 + k * 128, 128)])


def _sc_a(ent_pairs, wrd_pairs, eidx, widx):
    return pl.kernel(
        _sc_a_body,
        out_type=(jax.ShapeDtypeStruct((_A_G1, 2 * D), jnp.float32),
                  jax.ShapeDtypeStruct((_A_G2, 2 * D), jnp.float32)),
        mesh=_MESH,
        scratch_types=[
            pltpu.VMEM((_A_R,), jnp.int32),
            pltpu.VMEM((_A_R,), jnp.int32),
            pltpu.VMEM((_A_R, 2 * D), jnp.float32),
            pltpu.VMEM((_A_R, 2 * D), jnp.float32),
            pltpu.SemaphoreType.DMA,
            pltpu.SemaphoreType.DMA,
        ],
    )(ent_pairs, wrd_pairs, eidx, widx)


_B_S = 20
_B_G = 53248
_B_PW = _B_G // NWK
_B_CH = 16
_B_R = _B_CH * _B_S
_B_NCH = _B_PW // _B_CH


def _sc_b_body(tab_hbm, idx_hbm, out_hbm,
               idx0, idx1, rows0, rows1, outv, sem0, sem1):
    wid = _wid()
    bufs = ((idx0, rows0, sem0), (idx1, rows1, sem1))
    npw = _B_PW * _B_S
    slc = ((0, 128), (128, 128), (256, 64))

    def fire(c, par):
        idxb, rowsb, semb = bufs[par]
        pltpu.sync_copy(idx_hbm.at[pl.ds(wid * npw + c * _B_R, _B_R)], idxb)
        for o, n in slc:
            pltpu.async_copy(tab_hbm.at[idxb.at[pl.ds(o, n)]],
                             rowsb.at[pl.ds(o, n)], semb)

    def drain(par):
        idxb, rowsb, semb = bufs[par]
        for o, n in slc:
            pltpu.make_async_copy(tab_hbm.at[idxb.at[pl.ds(o, n)]],
                                  rowsb.at[pl.ds(o, n)], semb).wait()

    fire(0, 0)
    fire(1, 1)

    @pl.loop(0, _B_NCH, step=2)
    def _(c0):
        for par in range(2):
            c = c0 + par
            idxb, rowsb, semb = bufs[par]
            drain(par)

            def dosum(rowsb, outv, dd_lo, dd_hi):
                @pl.loop(0, _B_CH)
                def _(g):
                    r0 = g * _B_S
                    for dd in range(dd_lo, dd_hi):
                        sl = pl.ds(dd * 16, 16)
                        acc = rowsb[r0, sl]
                        for s in range(1, _B_S):
                            acc = acc + rowsb[r0 + s, sl]
                        outv[g, sl] = acc

            is_q = (wid * _B_PW + c * _B_CH) < B

            @pl.when(is_q)
            def _():
                dosum(rowsb, outv, 0, 4)

            @pl.when(jnp.logical_not(is_q))
            def _():
                dosum(rowsb, outv, 4, 8)

            pltpu.sync_copy(outv, out_hbm.at[pl.ds(wid * _B_PW + c * _B_CH, _B_CH)])
            nxt = c + 2

            @pl.when(nxt < _B_NCH)
            def _():
                fire(nxt, par)


def _sc_b(table, idx):
    return pl.kernel(
        _sc_b_body,
        out_type=jax.ShapeDtypeStruct((_B_G, 2 * D), jnp.float32),
        mesh=_MESH,
        scratch_types=[
            pltpu.VMEM((_B_R,), jnp.int32),
            pltpu.VMEM((_B_R,), jnp.int32),
            pltpu.VMEM((_B_R, 2 * D), jnp.float32),
            pltpu.VMEM((_B_R, 2 * D), jnp.float32),
            pltpu.VMEM((_B_CH, 2 * D), jnp.float32),
            pltpu.SemaphoreType.DMA,
            pltpu.SemaphoreType.DMA,
        ],
    )(table, idx)


_T1_RB = 2000


def _t1_body(w_ref, qW_ref, qb_ref, dW_ref, db_ref, out_ref):
    i = pl.program_id(0)
    x = w_ref[...]
    yq = jnp.tanh(jnp.dot(x, qW_ref[...], preferred_element_type=jnp.float32)
                  + qb_ref[0][None, :])
    yd = jnp.tanh(jnp.dot(x, dW_ref[...], preferred_element_type=jnp.float32)
                  + db_ref[0][None, :])
    y = jnp.concatenate([yq, yd], axis=1)
    row = lax.broadcasted_iota(jnp.int32, (_T1_RB, 1), 0) + i * _T1_RB
    out_ref[...] = jnp.where(row == 1, 0.0, y)


def _t1(word_emb, query_W, query_b, key_doc_W, key_doc_b):
    nblk = NWRD // _T1_RB
    return pl.pallas_call(
        _t1_body,
        grid=(nblk,),
        in_specs=[
            pl.BlockSpec((_T1_RB, D), lambda i: (i, 0)),
            pl.BlockSpec((D, D), lambda i: (0, 0)),
            pl.BlockSpec((1, D), lambda i: (0, 0)),
            pl.BlockSpec((D, D), lambda i: (0, 0)),
            pl.BlockSpec((1, D), lambda i: (0, 0)),
        ],
        out_specs=pl.BlockSpec((_T1_RB, 2 * D), lambda i: (i, 0)),
        out_shape=jax.ShapeDtypeStruct((NWRD, 2 * D), jnp.float32),
    )(word_emb, query_W, query_b[None, :], key_doc_W, key_doc_b[None, :])


def _t2_body(x_ref, par_ref, ids_ref, attW_ref, attv_ref, relW_ref, relb_ref,
             kkbW_ref, kkbb_ref, out_ref):
    xs = []
    cols = []
    for l in range(RL):
        x2 = x_ref[:, l, :]
        parl = par_ref[:, l][:, None]
        xl = jnp.where(parl == 1, x2[:, D:], x2[:, :D])
        xs.append(xl)
        tl = jnp.tanh(jnp.dot(xl, attW_ref[...], preferred_element_type=jnp.float32))
        cols.append(jnp.dot(tl, attv_ref[...], preferred_element_type=jnp.float32))
    s = jnp.concatenate(cols, axis=1)
    mask = ids_ref[...] != 1
    s = jnp.where(mask, s, -1e9)
    mx = jnp.max(s, axis=1, keepdims=True)
    e = jnp.exp(s - mx)
    a = e / jnp.sum(e, axis=1, keepdims=True)
    agg = jnp.zeros((NR, D), jnp.float32)
    for l in range(RL):
        agg = agg + a[:, l][:, None] * xs[l]
    rel_enc = jnp.tanh(jnp.dot(agg, relW_ref[...], preferred_element_type=jnp.float32)
                       + relb_ref[0][None, :])
    t2 = jnp.tanh(jnp.dot(rel_enc, kkbW_ref[...], preferred_element_type=jnp.float32)
                  + kkbb_ref[0][None, :])
    row = lax.broadcasted_iota(jnp.int32, (NR, 1), 0)
    out_ref[...] = jnp.where(row == 0, 0.0, t2)


def _t2(rel_rows, rel_par, rel_word_ids, att_W, att_v, rel_W, rel_b,
        key_kb_W, key_kb_b):
    return pl.pallas_call(
        _t2_body,
        out_shape=jax.ShapeDtypeStruct((NR, D), jnp.float32),
    )(rel_rows, rel_par, rel_word_ids, att_W, att_v[:, None], rel_W,
      rel_b[None, :], key_kb_W, key_kb_b[None, :])


_T3_RB = 6400


def _lrelu(x):
    return jnp.where(x >= 0, x, 0.01 * x)


def _psel(rows2, par):
    return jnp.where(par == 1, rows2[:, D:], rows2[:, :D])


_T3A_RB = 2048


def _t3a_body(sub_ref, tab_ref, idx_ref, par_ref, rel_ref, eW_ref, eb_ref,
              kW_ref, kb_ref, out_ref):
    x = _psel(sub_ref[...], par_ref[...])
    h = _lrelu(jnp.dot(x, eW_ref[...], preferred_element_type=jnp.float32)
               + eb_ref[0][None, :])
    h = jnp.tanh(jnp.dot(h, kW_ref[...], preferred_element_type=jnp.float32)
                 + kb_ref[0][None, :])
    mask = idx_ref[...] != 0
    col = lax.broadcasted_iota(jnp.int32, (_T3A_RB, NR), 1)
    oh = (col == rel_ref[...]).astype(jnp.float32)
    rel = jnp.dot(oh, tab_ref[...], preferred_element_type=jnp.float32)
    out_ref[...] = jnp.where(mask, h, 0.0) + rel


def _t3a(ent_rows, rel_table2, kk_sub_flat, par_flat, kk_rel_flat,
         eW, eb, kW, kb):
    grid = (B * M) // _T3A_RB
    bs = lambda i: (i, 0)
    return pl.pallas_call(
        _t3a_body,
        grid=(grid,),
        in_specs=[
            pl.BlockSpec((_T3A_RB, 2 * D), bs),
            pl.BlockSpec((NR, D), lambda i: (0, 0)),
            pl.BlockSpec((_T3A_RB, 1), bs),
            pl.BlockSpec((_T3A_RB, 1), bs),
            pl.BlockSpec((_T3A_RB, 1), bs),
            pl.BlockSpec((D, D), lambda i: (0, 0)),
            pl.BlockSpec((1, D), lambda i: (0, 0)),
            pl.BlockSpec((D, D), lambda i: (0, 0)),
            pl.BlockSpec((1, D), lambda i: (0, 0)),
        ],
        out_specs=pl.BlockSpec((_T3A_RB, D), bs),
        out_shape=jax.ShapeDtypeStruct((B * M, D), jnp.float32),
    )(ent_rows, rel_table2, kk_sub_flat, par_flat, kk_rel_flat, eW,
      eb[None, :], kW, kb[None, :])


def _t3b_body(v_ref, idx_ref, par_ref, W_ref, b_ref, out_ref):
    x = _psel(v_ref[...], par_ref[...])
    h = jnp.tanh(jnp.dot(x, W_ref[...], preferred_element_type=jnp.float32)
                 + b_ref[0][None, :])
    mask = idx_ref[...] != 0
    out_ref[...] = jnp.where(mask, h, 0.0)


def _t3b(ent_rows, off_blk, n, vidx_flat, par_flat, W, b):
    grid = n // _T3_RB
    return pl.pallas_call(
        _t3b_body,
        grid=(grid,),
        in_specs=[
            pl.BlockSpec((_T3_RB, 2 * D), lambda i: (i + off_blk, 0)),
            pl.BlockSpec((_T3_RB, 1), lambda i: (i, 0)),
            pl.BlockSpec((_T3_RB, 1), lambda i: (i, 0)),
            pl.BlockSpec((D, D), lambda i: (0, 0)),
            pl.BlockSpec((1, D), lambda i: (0, 0)),
        ],
        out_specs=pl.BlockSpec((_T3_RB, D), lambda i: (i, 0)),
        out_shape=jax.ShapeDtypeStruct((n, D), jnp.float32),
    )(ent_rows, vidx_flat, par_flat, W, b[None, :])


def _t3c_body(c_ref, par_ref, W_ref, b_ref, out_ref):
    x = _psel(c_ref[...], par_ref[...])
    out_ref[...] = _lrelu(jnp.dot(x, W_ref[...],
                                  preferred_element_type=jnp.float32)
                          + b_ref[0][None, :])


def _t3c(ent_rows, off_blk, n, par_flat, W, b):
    grid = n // _T3_RB
    return pl.pallas_call(
        _t3c_body,
        grid=(grid,),
        in_specs=[
            pl.BlockSpec((_T3_RB, 2 * D), lambda i: (i + off_blk, 0)),
            pl.BlockSpec((_T3_RB, 1), lambda i: (i, 0)),
            pl.BlockSpec((D, D), lambda i: (0, 0)),
            pl.BlockSpec((1, D), lambda i: (0, 0)),
        ],
        out_specs=pl.BlockSpec((_T3_RB, D), lambda i: (i, 0)),
        out_shape=jax.ShapeDtypeStruct((n, D), jnp.float32),
    )(ent_rows, par_flat, W, b[None, :])


_T4_BB = 64


def _t4_body(ekkb_ref, ekd_ref, evk_ref, evd_ref, ecand_ref, q_ref, ans_ref,
             oW_ref, ob_ref, lsum_ref, pred_ref, dist_ref):
    i = pl.program_id(0)
    q = q_ref[...]
    ekkb = ekkb_ref[...]
    ekd = ekd_ref[...]
    evk = evk_ref[...]
    evd = evd_ref[...]
    for _ in range(NUM_HOP):
        ph_kb = jnp.sum(ekkb * q[:, None, :], axis=2)
        ph_kd = jnp.sum(ekd * q[:, None, :], axis=2)
        ph_kb = jnp.where(ph_kb == 0.0, -1e9, ph_kb)
        ph_kd = jnp.where(ph_kd == 0.0, -1e9, ph_kd)
        mx = jnp.maximum(jnp.max(ph_kb, axis=1, keepdims=True),
                         jnp.max(ph_kd, axis=1, keepdims=True))
        e_kb = jnp.exp(ph_kb - mx)
        e_kd = jnp.exp(ph_kd - mx)
        tot = (jnp.sum(e_kb, axis=1, keepdims=True)
               + jnp.sum(e_kd, axis=1, keepdims=True))
        sc_kb = e_kb / tot
        sc_kd = e_kd / tot
        out = (jnp.sum(sc_kb[:, :, None] * evk, axis=1)
               + jnp.sum(sc_kd[:, :, None] * evd, axis=1))
        q = (jnp.dot(q + out, oW_ref[...], preferred_element_type=jnp.float32)
             + ob_ref[0][None, :])
    sp = jnp.sum(ecand_ref[...] * q[:, None, :], axis=2)
    y = ans_ref[...].astype(jnp.float32)
    contrib = jnp.sum(jnp.maximum(sp, 0.0) - sp * y
                      + jnp.log1p(jnp.exp(-jnp.abs(sp))))
    prev = jnp.where(i == 0, 0.0, lsum_ref[0, 0])
    lsum_ref[...] = jnp.full((1, 1), prev + contrib, dtype=jnp.float32)
    pred_ref[...] = jnp.argmax(sp, axis=1).astype(jnp.int32)[:, None]
    dist_ref[...] = 1.0 / (1.0 + jnp.exp(-sp))


def _t4(ekkb3, ekd3, evk3, evd3, ecand3, q0, answers, out_W, out_b):
    grid = B // _T4_BB
    return pl.pallas_call(
        _t4_body,
        grid=(grid,),
        in_specs=[
            pl.BlockSpec((_T4_BB, M, D), lambda i: (i, 0, 0)),
            pl.BlockSpec((_T4_BB, MD, D), lambda i: (i, 0, 0)),
            pl.BlockSpec((_T4_BB, M, D), lambda i: (i, 0, 0)),
            pl.BlockSpec((_T4_BB, MD, D), lambda i: (i, 0, 0)),
            pl.BlockSpec((_T4_BB, C, D), lambda i: (i, 0, 0)),
            pl.BlockSpec((_T4_BB, D), lambda i: (i, 0)),
            pl.BlockSpec((_T4_BB, C), lambda i: (i, 0)),
            pl.BlockSpec((D, D), lambda i: (0, 0)),
            pl.BlockSpec((1, D), lambda i: (0, 0)),
        ],
        out_specs=(
            pl.BlockSpec((1, 1), lambda i: (0, 0)),
            pl.BlockSpec((_T4_BB, 1), lambda i: (i, 0)),
            pl.BlockSpec((_T4_BB, C), lambda i: (i, 0)),
        ),
        out_shape=(
            jax.ShapeDtypeStruct((1, 1), jnp.float32),
            jax.ShapeDtypeStruct((B, 1), jnp.int32),
            jax.ShapeDtypeStruct((B, C), jnp.float32),
        ),
    )(ekkb3, ekd3, evk3, evd3, ecand3, q0, answers, out_W, out_b[None, :])


def kernel(questions, key_kb, rel_word_ids, key_doc, val_kb, val_doc,
           candidate_entities, answers, entity_emb, word_emb,
           entity_linear_W, entity_linear_b, att_W, att_v,
           rel_W, rel_b, query_W, query_b, key_kb_W, key_kb_b,
           key_doc_W, key_doc_b, value_W, value_b, out_W, out_b):
    i32 = jnp.int32
    kk_sub = key_kb[:, :, 0].reshape(-1).astype(i32)
    kk_rel = key_kb[:, :, 1].reshape(-1).astype(i32)
    vk = val_kb.reshape(-1).astype(i32)
    vd = val_doc.reshape(-1).astype(i32)
    cand = candidate_entities.reshape(-1).astype(i32)

    ent_pairs = entity_emb[:NENT].reshape(NENT // 2, 2 * D)
    wrd_pairs = word_emb.reshape(NWRD // 2, 2 * D)

    eidx = jnp.concatenate([
        kk_sub, vk, vd, cand,
        jnp.zeros((_A_G1 - 460800,), i32),
    ])
    widx = jnp.concatenate([
        rel_word_ids.reshape(-1).astype(i32),
        jnp.zeros((_A_G2 - NR * RL,), i32),
    ])
    ent_rows, wrd_rows = _sc_a(ent_pairs, wrd_pairs,
                               eidx >> 1, widx >> 1)

    word_tables = _t1(word_emb, query_W, query_b, key_doc_W, key_doc_b)

    rel_par = (rel_word_ids & 1).astype(i32)
    rel_table2 = _t2(wrd_rows[:NR * RL].reshape(NR, RL, 2 * D), rel_par,
                     rel_word_ids, att_W, att_v, rel_W, rel_b,
                     key_kb_W, key_kb_b)

    bidx = jnp.concatenate([
        questions.reshape(-1).astype(i32),
        key_doc.reshape(-1).astype(i32),
        jnp.zeros(((_B_G - B - B * MD) * _B_S,), i32),
    ])
    bsum = _sc_b(word_tables, bidx)
    q0 = bsum[:B, :D]
    ekd = bsum[B:B + B * MD, D:]

    ekkb = _t3a(ent_rows, rel_table2, kk_sub[:, None],
                (kk_sub & 1)[:, None], kk_rel[:, None],
                entity_linear_W, entity_linear_b, key_kb_W, key_kb_b)
    vall = jnp.concatenate([vk, vd])
    ev = _t3b(ent_rows, 102400 // _T3_RB, 153600, vall[:, None],
              (vall & 1)[:, None], value_W, value_b)
    ecand = _t3c(ent_rows, 256000 // _T3_RB, 204800,
                 (cand & 1)[:, None], entity_linear_W, entity_linear_b)

    lsum, pred, pred_dist = _t4(
        ekkb.reshape(B, M, D), ekd.reshape(B, MD, D),
        ev[:B * M].reshape(B, M, D), ev[B * M:].reshape(B, MD, D),
        ecand.reshape(B, C, D), q0, answers.astype(i32), out_W, out_b)
    loss = lsum[0, 0] / (B * C)
    return (loss, pred[:, 0], pred_dist)
